# Initial kernel scaffold; baseline (speedup 1.0000x reference)
import jax
import jax.numpy as jnp
from jax.experimental import pallas as pl


def kernel(x, edge_index, edge_weight, W1, b1, W2, b2, time_step_list):
    raise NotImplementedError("write your pallas kernel here")


# speedup vs baseline: 3.2620x; 3.2620x over previous
import functools

import jax
import jax.numpy as jnp
from jax import lax
from jax.experimental import pallas as pl
from jax.experimental.pallas import tpu as pltpu
from jax.experimental.pallas import tpu_sc as plsc

N = 10000
E = 320000
D = 128
H = 128
C = 64
L = 4

NC = 2
NS = 16
NW = NC * NS
EPT = E // NW
CHUNK = 128
NCHUNK = 80
EPT_PAD = NCHUNK * CHUNK
RPT = 624
LANES = 16


def _sc_propagate(h, src3, dst3, w3):
    mesh = plsc.VectorSubcoreMesh(core_axis_name="c", subcore_axis_name="s")

    @functools.partial(
        pl.kernel,
        mesh=mesh,
        out_type=jax.ShapeDtypeStruct((NC, N, H), jnp.float32),
        scratch_types=[
            pltpu.VMEM((NCHUNK, CHUNK), jnp.int32),
            pltpu.VMEM((NCHUNK, CHUNK), jnp.int32),
            pltpu.VMEM((NCHUNK, CHUNK), jnp.float32),
            pltpu.VMEM((CHUNK, H), jnp.float32),
            pltpu.VMEM_SHARED((N, H), jnp.float32),
            pltpu.SemaphoreType.DMA,
        ],
    )
    def k(h_hbm, src_hbm, dst_hbm, w_hbm, f_out, src_v, dst_v, w_v, buf,
          f_sh, sem):
        cid = lax.axis_index("c")
        sid = lax.axis_index("s")
        wid = sid * NC + cid
        last = sid == NS - 1

        pltpu.sync_copy(src_hbm.at[wid], src_v)
        pltpu.sync_copy(dst_hbm.at[wid], dst_v)
        pltpu.sync_copy(w_hbm.at[wid], w_v)

        zeros = jnp.zeros((LANES,), jnp.float32)

        def zrow(r, carry):
            for j in range(H // LANES):
                buf[r, pl.ds(j * LANES, LANES)] = zeros
            return carry

        lax.fori_loop(0, CHUNK, zrow, 0)
        base = sid * RPT
        for z in range(4):
            pltpu.sync_copy(buf, f_sh.at[pl.ds(base + z * CHUNK, CHUNK)])

        @pl.when(last)
        def _():
            pltpu.sync_copy(buf, f_sh.at[pl.ds(base + 4 * CHUNK, CHUNK)])

        @pl.when(jnp.logical_not(last))
        def _():
            pltpu.sync_copy(buf.at[pl.ds(0, RPT - 4 * CHUNK)],
                            f_sh.at[pl.ds(base + 4 * CHUNK, RPT - 4 * CHUNK)])

        plsc.subcore_barrier()

        def chunk_body(ci, carry):
            pltpu.async_copy(h_hbm.at[src_v.at[ci]], buf, sem).wait()

            def group_body(g, c2):
                wvec = w_v[ci, pl.ds(g * LANES, LANES)]
                for lane in range(LANES):
                    we = wvec[lane]
                    e = g * LANES + lane
                    for j in range(H // LANES):
                        sl = pl.ds(j * LANES, LANES)
                        buf[e, sl] = buf[e, sl] * we
                return c2

            lax.fori_loop(0, CHUNK // LANES, group_body, 0)
            pltpu.sync_copy(buf, f_sh.at[dst_v.at[ci]], add=True)
            return carry

        lax.fori_loop(0, NCHUNK, chunk_body, 0)
        plsc.subcore_barrier()

        for z in range(4):
            pltpu.sync_copy(f_sh.at[pl.ds(base + z * CHUNK, CHUNK)],
                            f_out.at[cid, pl.ds(base + z * CHUNK, CHUNK)])

        @pl.when(last)
        def _():
            pltpu.sync_copy(f_sh.at[pl.ds(base + 4 * CHUNK, CHUNK)],
                            f_out.at[cid, pl.ds(base + 4 * CHUNK, CHUNK)])

        @pl.when(jnp.logical_not(last))
        def _():
            r = RPT - 4 * CHUNK
            pltpu.sync_copy(f_sh.at[pl.ds(base + 4 * CHUNK, r)],
                            f_out.at[cid, pl.ds(base + 4 * CHUNK, r)])

    return k(h, src3, dst3, w3)


def _tc_entry(x, w1t, b1):
    def body(x_ref, w_ref, b_ref, o_ref):
        acc = jnp.dot(x_ref[...], w_ref[...],
                      preferred_element_type=jnp.float32)
        o_ref[...] = jnp.maximum(acc + b_ref[...], 0.0)

    return pl.pallas_call(
        body,
        grid=(10,),
        in_specs=[
            pl.BlockSpec((N // 10, D), lambda i: (i, 0)),
            pl.BlockSpec((D, H), lambda i: (0, 0)),
            pl.BlockSpec((1, H), lambda i: (0, 0)),
        ],
        out_specs=pl.BlockSpec((N // 10, H), lambda i: (i, 0)),
        out_shape=jax.ShapeDtypeStruct((N, H), jnp.float32),
    )(x, w1t, b1.reshape(1, H))


def _tc_combine(h, f0, f1, dt):
    def body(h_ref, f0_ref, f1_ref, dt_ref, o_ref):
        f = jnp.maximum(f0_ref[...] + f1_ref[...], 0.0)
        o_ref[...] = h_ref[...] + f * dt_ref[0]

    blk = pl.BlockSpec((N // 10, H), lambda i: (i, 0))
    return pl.pallas_call(
        body,
        grid=(10,),
        in_specs=[
            blk, blk, blk,
            pl.BlockSpec(memory_space=pltpu.SMEM),
        ],
        out_specs=blk,
        out_shape=jax.ShapeDtypeStruct((N, H), jnp.float32),
    )(h, f0, f1, dt)


def _tc_final(h, f0, f1, dt, w2t, b2):
    def body(h_ref, f0_ref, f1_ref, dt_ref, w_ref, b_ref, o_ref):
        f = jnp.maximum(f0_ref[...] + f1_ref[...], 0.0)
        hh = h_ref[...] + f * dt_ref[0]
        acc = jnp.dot(hh, w_ref[...], preferred_element_type=jnp.float32)
        o_ref[...] = acc + b_ref[...]

    blk = pl.BlockSpec((N // 10, H), lambda i: (i, 0))
    return pl.pallas_call(
        body,
        grid=(10,),
        in_specs=[
            blk, blk, blk,
            pl.BlockSpec(memory_space=pltpu.SMEM),
            pl.BlockSpec((H, C), lambda i: (0, 0)),
            pl.BlockSpec((1, C), lambda i: (0, 0)),
        ],
        out_specs=pl.BlockSpec((N // 10, C), lambda i: (i, 0)),
        out_shape=jax.ShapeDtypeStruct((N, C), jnp.float32),
    )(h, f0, f1, dt, w2t, b2.reshape(1, C))


def kernel(x, edge_index, edge_weight, W1, b1, W2, b2, time_step_list):
    pad = ((0, 0), (0, EPT_PAD - EPT))
    dst3 = jnp.pad(edge_index[0].reshape(NW, EPT), pad).reshape(
        NW, NCHUNK, CHUNK)
    src3 = jnp.pad(edge_index[1].reshape(NW, EPT), pad).reshape(
        NW, NCHUNK, CHUNK)
    w3 = jnp.pad(edge_weight.reshape(NW, EPT), pad).reshape(
        NW, NCHUNK, CHUNK)

    h = _tc_entry(x, W1.T, b1)
    out = None
    for i in range(L):
        f = _sc_propagate(h, src3, dst3, w3)
        dt = time_step_list[i].reshape(1)
        if i < L - 1:
            h = _tc_combine(h, f[0], f[1], dt)
        else:
            out = _tc_final(h, f[0], f[1], dt, W2.T, b2)
    return out
